# baseline (speedup 1.0000x reference)
import functools

import jax
import jax.numpy as jnp
from jax import lax
from jax.experimental import pallas as pl
from jax.experimental.pallas import tpu as pltpu
from jax.experimental.pallas import tpu_sc as plsc

_C = 1.0
_NC = 2
_NS = 16
_L = 16
_SW = 8
_CE = 4096
_STRIP = _CE // _NS


def _node_table_body(x_ref, wp_ref, b_ref, tab_ref):
    x = x_ref[...]
    x0 = x[:, 0:1]
    total = jnp.sum(x * x, axis=1, keepdims=True)
    nsq = jnp.maximum(total - x0 * x0, 0.0)
    norm = jnp.maximum(jnp.sqrt(nsq), 1e-8)
    x0c = jnp.maximum(x0, 1.0 + 1e-6)
    dist = jnp.log(x0c + jnp.sqrt((x0c - 1.0) * (x0c + 1.0)))
    s = dist / norm
    mu = jnp.dot(x, wp_ref[...], preferred_element_type=jnp.float32) * s
    mu = mu + b_ref[0:1, :]
    mn = jnp.maximum(jnp.sqrt(jnp.sum(mu * mu, axis=1, keepdims=True)), 1e-8)
    e = jnp.exp(mn)
    ei = 1.0 / e
    ch = 0.5 * (e + ei)
    sh = 0.5 * (e - ei)
    tail = (sh / mn) * mu
    used = tail[:, : x.shape[1] - 1]
    tab = jnp.concatenate([ch, used], axis=1)
    for t in range(_NS):
        tab_ref[t] = tab[:, t * _SW:(t + 1) * _SW]


def _make_node_table(x, wp, b8, bn):
    n, d_in = x.shape
    grid = n // bn
    return pl.pallas_call(
        _node_table_body,
        grid=(grid,),
        in_specs=[
            pl.BlockSpec((bn, d_in), lambda i: (i, 0)),
            pl.BlockSpec((d_in, d_in), lambda i: (0, 0)),
            pl.BlockSpec((8, d_in), lambda i: (0, 0)),
        ],
        out_specs=pl.BlockSpec((_NS, bn, _SW), lambda i: (0, i, 0)),
        out_shape=jax.ShapeDtypeStruct((_NS, n, _SW), jnp.float32),
    )(x, wp, b8)


def _make_edge_kernel(e_total, n_rows):
    mesh = plsc.VectorSubcoreMesh(
        core_axis_name="c", subcore_axis_name="s", num_cores=_NC
    )
    epc = e_total // _NC
    nch = epc // _CE
    groups = _CE // _L
    slab_words = n_rows * _SW

    @functools.partial(
        pl.kernel,
        mesh=mesh,
        compiler_params=pltpu.CompilerParams(needs_layout_passes=False),
        out_type=jax.ShapeDtypeStruct((e_total,), jnp.float32),
        scratch_types=[
            pltpu.VMEM((slab_words,), jnp.float32),
            pltpu.VMEM((2 * _CE,), jnp.int32),
            pltpu.VMEM((2 * _CE,), jnp.int32),
            pltpu.VMEM((_CE,), jnp.float32),
            pltpu.VMEM((_NS, _STRIP), jnp.float32),
            pltpu.VMEM((_STRIP,), jnp.float32),
            pltpu.VMEM_SHARED((_NS, _CE), jnp.float32),
            pltpu.VMEM_SHARED((_NS, _CE), jnp.float32),
            pltpu.SemaphoreType.DMA,
            pltpu.SemaphoreType.DMA,
        ],
    )
    def edge_kernel(tabt_hbm, src_hbm, dst_hbm, out_hbm,
                    slab_v, idx0, idx1, part_v, sum_v, res_v,
                    xch0, xch1, sem_i0, sem_i1):
        cid = lax.axis_index("c")
        sid = lax.axis_index("s")
        iota = lax.iota(jnp.int32, _L)
        sgn0 = jnp.where(sid == 0, -1.0, 1.0)
        sgn = jnp.zeros((_L,), jnp.float32) + sgn0

        pltpu.sync_copy(tabt_hbm.at[sid], slab_v)

        edge_base = cid * nch * _CE
        out_base = cid * nch * _CE

        def idx_descs(c, ibuf, isem):
            off = edge_base + c * _CE
            return (
                pltpu.make_async_copy(src_hbm.at[pl.ds(off, _CE)],
                                      ibuf.at[pl.ds(0, _CE)], isem),
                pltpu.make_async_copy(dst_hbm.at[pl.ds(off, _CE)],
                                      ibuf.at[pl.ds(_CE, _CE)], isem),
            )

        def idx_start(c, ibuf, isem):
            d1, d2 = idx_descs(c, ibuf, isem)
            d1.start()
            d2.start()

        def idx_wait(c, ibuf, isem):
            d1, d2 = idx_descs(c, ibuf, isem)
            d1.wait()
            d2.wait()

        idx_start(0, idx0, sem_i0)
        idx_wait(0, idx0, sem_i0)

        @pl.when(nch > 1)
        def _():
            idx_start(1, idx1, sem_i1)

        def process_chunk(c, ibuf, isem, xch):
            @pl.when(c > 0)
            def _():
                idx_wait(c, ibuf, isem)

            @plsc.parallel_loop(0, _CE, _L, unroll=4)
            def group_body(goff):
                sidx = ibuf[pl.ds(goff, _L)]
                didx = ibuf[pl.ds(_CE + goff, _L)]
                rs = sidx * _SW
                rd = didx * _SW
                av = [plsc.load_gather(slab_v, [rs + d] if d else [rs])
                      for d in range(_SW)]
                bv = [plsc.load_gather(slab_v, [rd + d] if d else [rd])
                      for d in range(_SW)]
                prods = [(av[0] * sgn) * bv[0]]
                prods += [av[d] * bv[d] for d in range(1, _SW)]
                while len(prods) > 1:
                    prods = [prods[i] + prods[i + 1]
                             for i in range(0, len(prods) - 1, 2)] + (
                                 [prods[-1]] if len(prods) % 2 else [])
                part_v[pl.ds(goff, _L)] = prods[0]

            pltpu.sync_copy(part_v, xch.at[sid])
            plsc.subcore_barrier()
            pltpu.sync_copy(xch.at[:, pl.ds(sid * _STRIP, _STRIP)], sum_v)

            for v in range(_STRIP // _L):
                tot = sum_v[0, pl.ds(v * _L, _L)]
                for r in range(1, _NS):
                    tot = tot + sum_v[r, pl.ds(v * _L, _L)]
                t = -_C - tot
                rr = jnp.minimum(jnp.maximum(t, 1e-10), 1.0)
                res_v[pl.ds(v * _L, _L)] = jnp.exp(-rr)
            pltpu.sync_copy(
                res_v,
                out_hbm.at[pl.ds(out_base + c * _CE + sid * _STRIP, _STRIP)])

            @pl.when(c < nch - 2)
            def _():
                idx_start(c + 2, ibuf, isem)

        def chunk_body(c, carry):
            @pl.when(lax.rem(c, 2) == 0)
            def _even():
                process_chunk(c, idx0, sem_i0, xch0)

            @pl.when(lax.rem(c, 2) == 1)
            def _odd():
                process_chunk(c, idx1, sem_i1, xch1)

            return carry

        lax.fori_loop(0, nch, chunk_body, 0)

    return edge_kernel


def kernel(x, edge_index, W, b):
    n, d_in = x.shape
    e = edge_index.shape[1]
    x = x.astype(jnp.float32)
    wp = jnp.concatenate(
        [jnp.zeros((1, d_in), jnp.float32), W.astype(jnp.float32)], axis=0
    )
    b8 = jnp.broadcast_to(b.astype(jnp.float32), (8, d_in))

    bn = 256
    n_pad = ((n + bn - 1) // bn) * bn
    xp = x if n_pad == n else jnp.pad(x, ((0, n_pad - n), (0, 0)))
    tabt = _make_node_table(xp, wp, b8, bn).reshape(
        _NS, n_pad * _SW)

    src = edge_index[0].astype(jnp.int32)
    dst = edge_index[1].astype(jnp.int32)
    chunk = _NC * _CE
    e_pad = ((e + chunk - 1) // chunk) * chunk
    if e_pad != e:
        src = jnp.pad(src, (0, e_pad - e))
        dst = jnp.pad(dst, (0, e_pad - e))

    res = _make_edge_kernel(e_pad, n_pad)(tabt, src, dst)
    if e_pad != e:
        res = res[:e]
    return (edge_index, res, (n, n))


# speedup vs baseline: 1.1740x; 1.1740x over previous
import functools

import jax
import jax.numpy as jnp
from jax import lax
from jax.experimental import pallas as pl
from jax.experimental.pallas import tpu as pltpu
from jax.experimental.pallas import tpu_sc as plsc

_C = 1.0
_NC = 2
_NS = 16
_L = 16
_SW = 8
_CE = 4096
_STRIP = _CE // _NS


def _node_table_body(x_ref, wp_ref, b_ref, tab_ref):
    x = x_ref[...]
    x0 = x[:, 0:1]
    total = jnp.sum(x * x, axis=1, keepdims=True)
    nsq = jnp.maximum(total - x0 * x0, 0.0)
    norm = jnp.maximum(jnp.sqrt(nsq), 1e-8)
    x0c = jnp.maximum(x0, 1.0 + 1e-6)
    dist = jnp.log(x0c + jnp.sqrt((x0c - 1.0) * (x0c + 1.0)))
    s = dist / norm
    mu = jnp.dot(x, wp_ref[...], preferred_element_type=jnp.float32) * s
    mu = mu + b_ref[0:1, :]
    mn = jnp.maximum(jnp.sqrt(jnp.sum(mu * mu, axis=1, keepdims=True)), 1e-8)
    e = jnp.exp(mn)
    ei = 1.0 / e
    ch = 0.5 * (e + ei)
    sh = 0.5 * (e - ei)
    tail = (sh / mn) * mu
    used = tail[:, : x.shape[1] - 1]
    tab_ref[...] = jnp.concatenate([ch, used], axis=1)


def _make_node_table(x, wp, b8, bn):
    n, d_in = x.shape
    grid = n // bn
    return pl.pallas_call(
        _node_table_body,
        grid=(grid,),
        in_specs=[
            pl.BlockSpec((bn, d_in), lambda i: (i, 0)),
            pl.BlockSpec((d_in, d_in), lambda i: (0, 0)),
            pl.BlockSpec((8, d_in), lambda i: (0, 0)),
        ],
        out_specs=pl.BlockSpec((bn, d_in), lambda i: (i, 0)),
        out_shape=jax.ShapeDtypeStruct((n, d_in), jnp.float32),
    )(x, wp, b8)


def _make_edge_kernel(e_total, n_rows):
    mesh = plsc.VectorSubcoreMesh(
        core_axis_name="c", subcore_axis_name="s", num_cores=_NC
    )
    epc = e_total // _NC
    nch = epc // _CE
    groups = _CE // _L
    slab_words = n_rows * _SW

    @functools.partial(
        pl.kernel,
        mesh=mesh,
        compiler_params=pltpu.CompilerParams(needs_layout_passes=False),
        out_type=jax.ShapeDtypeStruct((e_total,), jnp.float32),
        scratch_types=[
            pltpu.VMEM((slab_words,), jnp.float32),
            pltpu.VMEM((2 * _CE,), jnp.int32),
            pltpu.VMEM((2 * _CE,), jnp.int32),
            pltpu.VMEM((_CE,), jnp.float32),
            pltpu.VMEM((_NS, _STRIP), jnp.float32),
            pltpu.VMEM((_STRIP,), jnp.float32),
            pltpu.VMEM_SHARED((_NS, _CE), jnp.float32),
            pltpu.VMEM_SHARED((_NS, _CE), jnp.float32),
            pltpu.SemaphoreType.DMA,
            pltpu.SemaphoreType.DMA,
            pltpu.SemaphoreType.DMA,
        ],
    )
    def edge_kernel(tabt_hbm, src_hbm, dst_hbm, out_hbm,
                    slab_v, idx0, idx1, part_v, sum_v, res_v,
                    xch0, xch1, sem_i0, sem_i1, sem_x):
        cid = lax.axis_index("c")
        sid = lax.axis_index("s")
        iota = lax.iota(jnp.int32, _L)
        sgn0 = jnp.where(sid == 0, -1.0, 1.0)
        sgn = jnp.zeros((_L,), jnp.float32) + sgn0

        pltpu.sync_copy(tabt_hbm.at[sid], slab_v)

        edge_base = cid * nch * _CE
        out_base = cid * nch * _CE

        def idx_descs(c, ibuf, isem):
            off = edge_base + c * _CE
            return (
                pltpu.make_async_copy(src_hbm.at[pl.ds(off, _CE)],
                                      ibuf.at[pl.ds(0, _CE)], isem),
                pltpu.make_async_copy(dst_hbm.at[pl.ds(off, _CE)],
                                      ibuf.at[pl.ds(_CE, _CE)], isem),
            )

        def idx_start(c, ibuf, isem):
            d1, d2 = idx_descs(c, ibuf, isem)
            d1.start()
            d2.start()

        def idx_wait(c, ibuf, isem):
            d1, d2 = idx_descs(c, ibuf, isem)
            d1.wait()
            d2.wait()

        idx_start(0, idx0, sem_i0)
        idx_wait(0, idx0, sem_i0)

        @pl.when(nch > 1)
        def _():
            idx_start(1, idx1, sem_i1)

        def reduce_chunk(c, xch):
            pltpu.sync_copy(xch.at[:, pl.ds(sid * _STRIP, _STRIP)], sum_v)
            for v in range(_STRIP // _L):
                tot = sum_v[0, pl.ds(v * _L, _L)]
                for r in range(1, _NS):
                    tot = tot + sum_v[r, pl.ds(v * _L, _L)]
                t = -_C - tot
                rr = jnp.minimum(jnp.maximum(t, 1e-10), 1.0)
                res_v[pl.ds(v * _L, _L)] = jnp.exp(-rr)
            pltpu.sync_copy(
                res_v,
                out_hbm.at[pl.ds(out_base + c * _CE + sid * _STRIP, _STRIP)])

        def process_chunk(c, ibuf, isem, xch, xch_prev):
            @pl.when(c > 0)
            def _():
                idx_wait(c, ibuf, isem)

            @plsc.parallel_loop(0, _CE, _L, unroll=4)
            def group_body(goff):
                sidx = ibuf[pl.ds(goff, _L)]
                didx = ibuf[pl.ds(_CE + goff, _L)]
                rs = sidx * _SW
                rd = didx * _SW
                av = [plsc.load_gather(slab_v, [rs + d] if d else [rs])
                      for d in range(_SW)]
                bv = [plsc.load_gather(slab_v, [rd + d] if d else [rd])
                      for d in range(_SW)]
                prods = [(av[0] * sgn) * bv[0]]
                prods += [av[d] * bv[d] for d in range(1, _SW)]
                while len(prods) > 1:
                    prods = [prods[i] + prods[i + 1]
                             for i in range(0, len(prods) - 1, 2)] + (
                                 [prods[-1]] if len(prods) % 2 else [])
                part_v[pl.ds(goff, _L)] = prods[0]

            xdesc = pltpu.make_async_copy(part_v, xch.at[sid], sem_x)
            xdesc.start()

            @pl.when(c > 0)
            def _():
                reduce_chunk(c - 1, xch_prev)

            @pl.when(c < nch - 2)
            def _():
                idx_start(c + 2, ibuf, isem)

            xdesc.wait()
            plsc.subcore_barrier()

        def chunk_body(c, carry):
            @pl.when(lax.rem(c, 2) == 0)
            def _even():
                process_chunk(c, idx0, sem_i0, xch0, xch1)

            @pl.when(lax.rem(c, 2) == 1)
            def _odd():
                process_chunk(c, idx1, sem_i1, xch1, xch0)

            return carry

        lax.fori_loop(0, nch, chunk_body, 0)
        reduce_chunk(nch - 1, xch1 if (nch - 1) % 2 else xch0)

    return edge_kernel


def kernel(x, edge_index, W, b):
    n, d_in = x.shape
    e = edge_index.shape[1]
    x = x.astype(jnp.float32)
    wp = jnp.concatenate(
        [jnp.zeros((1, d_in), jnp.float32), W.astype(jnp.float32)], axis=0
    )
    b8 = jnp.broadcast_to(b.astype(jnp.float32), (8, d_in))

    bn = 256
    n_pad = ((n + bn - 1) // bn) * bn
    xp = x if n_pad == n else jnp.pad(x, ((0, n_pad - n), (0, 0)))
    tab = _make_node_table(xp, wp, b8, bn)
    tabt = tab.reshape(n_pad, _NS, _SW).transpose(1, 0, 2).reshape(
        _NS, n_pad * _SW)

    src = edge_index[0].astype(jnp.int32)
    dst = edge_index[1].astype(jnp.int32)
    chunk = _NC * _CE
    e_pad = ((e + chunk - 1) // chunk) * chunk
    if e_pad != e:
        src = jnp.pad(src, (0, e_pad - e))
        dst = jnp.pad(dst, (0, e_pad - e))

    res = _make_edge_kernel(e_pad, n_pad)(tabt, src, dst)
    if e_pad != e:
        res = res[:e]
    return (edge_index, res, (n, n))


# speedup vs baseline: 1.1748x; 1.0007x over previous
import functools

import jax
import jax.numpy as jnp
from jax import lax
from jax.experimental import pallas as pl
from jax.experimental.pallas import tpu as pltpu
from jax.experimental.pallas import tpu_sc as plsc

_C = 1.0
_NC = 2
_NS = 16
_L = 16
_SW = 8
_CE = 4096
_STRIP = _CE // _NS


def _node_table_body(x_ref, wp_ref, b_ref, tab_ref):
    x = x_ref[...]
    x0 = x[:, 0:1]
    total = jnp.sum(x * x, axis=1, keepdims=True)
    nsq = jnp.maximum(total - x0 * x0, 0.0)
    norm = jnp.maximum(jnp.sqrt(nsq), 1e-8)
    x0c = jnp.maximum(x0, 1.0 + 1e-6)
    dist = jnp.log(x0c + jnp.sqrt((x0c - 1.0) * (x0c + 1.0)))
    s = dist / norm
    mu = jnp.dot(x, wp_ref[...], preferred_element_type=jnp.float32) * s
    mu = mu + b_ref[0:1, :]
    mn = jnp.maximum(jnp.sqrt(jnp.sum(mu * mu, axis=1, keepdims=True)), 1e-8)
    e = jnp.exp(mn)
    ei = 1.0 / e
    ch = 0.5 * (e + ei)
    sh = 0.5 * (e - ei)
    tail = (sh / mn) * mu
    used = tail[:, : x.shape[1] - 1]
    tab_ref[...] = jnp.concatenate([ch, used], axis=1)


def _make_node_table(x, wp, b8, bn):
    n, d_in = x.shape
    grid = n // bn
    return pl.pallas_call(
        _node_table_body,
        grid=(grid,),
        in_specs=[
            pl.BlockSpec((bn, d_in), lambda i: (i, 0)),
            pl.BlockSpec((d_in, d_in), lambda i: (0, 0)),
            pl.BlockSpec((8, d_in), lambda i: (0, 0)),
        ],
        out_specs=pl.BlockSpec((bn, d_in), lambda i: (i, 0)),
        out_shape=jax.ShapeDtypeStruct((n, d_in), jnp.float32),
    )(x, wp, b8)


def _make_edge_kernel(e_total, n_rows):
    mesh = plsc.VectorSubcoreMesh(
        core_axis_name="c", subcore_axis_name="s", num_cores=_NC
    )
    epc = e_total // _NC
    nch = epc // _CE
    groups = _CE // _L
    slab_words = n_rows * _SW

    @functools.partial(
        pl.kernel,
        mesh=mesh,
        compiler_params=pltpu.CompilerParams(
            needs_layout_passes=False, disable_bounds_checks=True),
        out_type=jax.ShapeDtypeStruct((e_total,), jnp.float32),
        scratch_types=[
            pltpu.VMEM((slab_words,), jnp.float32),
            pltpu.VMEM((2 * _CE,), jnp.int32),
            pltpu.VMEM((2 * _CE,), jnp.int32),
            pltpu.VMEM((_CE,), jnp.float32),
            pltpu.VMEM((_NS, _STRIP), jnp.float32),
            pltpu.VMEM((_STRIP,), jnp.float32),
            pltpu.VMEM_SHARED((_NS, _CE), jnp.float32),
            pltpu.VMEM_SHARED((_NS, _CE), jnp.float32),
            pltpu.SemaphoreType.DMA,
            pltpu.SemaphoreType.DMA,
            pltpu.SemaphoreType.DMA,
        ],
    )
    def edge_kernel(tabt_hbm, src_hbm, dst_hbm, out_hbm,
                    slab_v, idx0, idx1, part_v, sum_v, res_v,
                    xch0, xch1, sem_i0, sem_i1, sem_x):
        cid = lax.axis_index("c")
        sid = lax.axis_index("s")
        iota = lax.iota(jnp.int32, _L)
        sgn0 = jnp.where(sid == 0, -1.0, 1.0)
        sgn = jnp.zeros((_L,), jnp.float32) + sgn0

        pltpu.sync_copy(tabt_hbm.at[sid], slab_v)

        edge_base = cid * nch * _CE
        out_base = cid * nch * _CE

        def idx_descs(c, ibuf, isem):
            off = edge_base + c * _CE
            return (
                pltpu.make_async_copy(src_hbm.at[pl.ds(off, _CE)],
                                      ibuf.at[pl.ds(0, _CE)], isem),
                pltpu.make_async_copy(dst_hbm.at[pl.ds(off, _CE)],
                                      ibuf.at[pl.ds(_CE, _CE)], isem),
            )

        def idx_start(c, ibuf, isem):
            d1, d2 = idx_descs(c, ibuf, isem)
            d1.start()
            d2.start()

        def idx_wait(c, ibuf, isem):
            d1, d2 = idx_descs(c, ibuf, isem)
            d1.wait()
            d2.wait()

        idx_start(0, idx0, sem_i0)
        idx_wait(0, idx0, sem_i0)

        @pl.when(nch > 1)
        def _():
            idx_start(1, idx1, sem_i1)

        def reduce_chunk(c, xch):
            pltpu.sync_copy(xch.at[:, pl.ds(sid * _STRIP, _STRIP)], sum_v)
            for v in range(_STRIP // _L):
                tot = sum_v[0, pl.ds(v * _L, _L)]
                for r in range(1, _NS):
                    tot = tot + sum_v[r, pl.ds(v * _L, _L)]
                t = -_C - tot
                rr = jnp.minimum(jnp.maximum(t, 1e-10), 1.0)
                res_v[pl.ds(v * _L, _L)] = jnp.exp(-rr)
            pltpu.sync_copy(
                res_v,
                out_hbm.at[pl.ds(out_base + c * _CE + sid * _STRIP, _STRIP)])

        def process_chunk(c, ibuf, isem, xch, xch_prev):
            @pl.when(c > 0)
            def _():
                idx_wait(c, ibuf, isem)

            @plsc.parallel_loop(0, _CE, _L, unroll=4)
            def group_body(goff):
                sidx = ibuf[pl.ds(goff, _L)]
                didx = ibuf[pl.ds(_CE + goff, _L)]
                rs = sidx * _SW
                rd = didx * _SW
                av = [plsc.load_gather(slab_v, [rs + d] if d else [rs])
                      for d in range(_SW)]
                bv = [plsc.load_gather(slab_v, [rd + d] if d else [rd])
                      for d in range(_SW)]
                prods = [(av[0] * sgn) * bv[0]]
                prods += [av[d] * bv[d] for d in range(1, _SW)]
                while len(prods) > 1:
                    prods = [prods[i] + prods[i + 1]
                             for i in range(0, len(prods) - 1, 2)] + (
                                 [prods[-1]] if len(prods) % 2 else [])
                part_v[pl.ds(goff, _L)] = prods[0]

            xdesc = pltpu.make_async_copy(part_v, xch.at[sid], sem_x)
            xdesc.start()

            @pl.when(c > 0)
            def _():
                reduce_chunk(c - 1, xch_prev)

            @pl.when(c < nch - 2)
            def _():
                idx_start(c + 2, ibuf, isem)

            xdesc.wait()
            plsc.subcore_barrier()

        def chunk_body(c, carry):
            @pl.when(lax.rem(c, 2) == 0)
            def _even():
                process_chunk(c, idx0, sem_i0, xch0, xch1)

            @pl.when(lax.rem(c, 2) == 1)
            def _odd():
                process_chunk(c, idx1, sem_i1, xch1, xch0)

            return carry

        lax.fori_loop(0, nch, chunk_body, 0)
        reduce_chunk(nch - 1, xch1 if (nch - 1) % 2 else xch0)

    return edge_kernel


def kernel(x, edge_index, W, b):
    n, d_in = x.shape
    e = edge_index.shape[1]
    x = x.astype(jnp.float32)
    wp = jnp.concatenate(
        [jnp.zeros((1, d_in), jnp.float32), W.astype(jnp.float32)], axis=0
    )
    b8 = jnp.broadcast_to(b.astype(jnp.float32), (8, d_in))

    bn = 256
    n_pad = ((n + bn - 1) // bn) * bn
    xp = x if n_pad == n else jnp.pad(x, ((0, n_pad - n), (0, 0)))
    tab = _make_node_table(xp, wp, b8, bn)
    tabt = tab.reshape(n_pad, _NS, _SW).transpose(1, 0, 2).reshape(
        _NS, n_pad * _SW)

    src = edge_index[0].astype(jnp.int32)
    dst = edge_index[1].astype(jnp.int32)
    chunk = _NC * _CE
    e_pad = ((e + chunk - 1) // chunk) * chunk
    if e_pad != e:
        src = jnp.pad(src, (0, e_pad - e))
        dst = jnp.pad(dst, (0, e_pad - e))

    res = _make_edge_kernel(e_pad, n_pad)(tabt, src, dst)
    if e_pad != e:
        res = res[:e]
    return (edge_index, res, (n, n))
